# Initial kernel scaffold; baseline (speedup 1.0000x reference)
import jax
import jax.numpy as jnp
from jax.experimental import pallas as pl


def kernel(code_bytes, embedding, positional):
    raise NotImplementedError("write your pallas kernel here")


# speedup vs baseline: 1.7532x; 1.7532x over previous
import functools

import jax
import jax.numpy as jnp
from jax import lax
from jax.experimental import pallas as pl
from jax.experimental.pallas import tpu as pltpu
from jax.experimental.pallas import tpu_sc as plsc

VOCAB = 256
D = 128
SEQ = 200
CH = 128
LANES = 16


def _embed_kernel(T, NC, NS):
    NW = NC * NS
    tok_per_w = T // NW
    nchunk = tok_per_w // CH
    mesh = plsc.VectorSubcoreMesh(core_axis_name="c", subcore_axis_name="s")

    @functools.partial(
        pl.kernel,
        mesh=mesh,
        out_type=jax.ShapeDtypeStruct((T, D), jnp.float32),
        scratch_types=[
            pltpu.VMEM((CH,), jnp.int32),
            pltpu.VMEM((CH, D), jnp.float32),
            pltpu.VMEM((2 * SEQ, D), jnp.float32),
            pltpu.SemaphoreType.DMA,
        ],
    )
    def k(idx_hbm, pos2_hbm, table_hbm, out_hbm, idx_v, rows_v, pos2_v, sem):
        c = lax.axis_index("c")
        s = lax.axis_index("s")
        wid = s * NC + c
        base = wid * tok_per_w
        pltpu.sync_copy(pos2_hbm, pos2_v)

        def chunk_body(i, carry):
            tok = base + i * CH
            pltpu.sync_copy(idx_hbm.at[pl.ds(tok, CH)], idx_v)
            pltpu.async_copy(table_hbm.at[idx_v], rows_v, sem).wait()
            p0 = lax.rem(i * CH, SEQ)

            def row_body(r, rc):
                pr = p0 + r
                for j in range(D // LANES):
                    sl = pl.ds(j * LANES, LANES)
                    rows_v[r, sl] = rows_v[r, sl] + pos2_v[pr, sl]
                return rc

            lax.fori_loop(0, CH, row_body, 0)
            pltpu.sync_copy(rows_v, out_hbm.at[pl.ds(tok, CH)])
            return carry

        lax.fori_loop(0, nchunk, chunk_body, 0)

    return k


def kernel(code_bytes, embedding, positional):
    batch, seq = code_bytes.shape
    idx_flat = code_bytes.reshape(-1).astype(jnp.int32)
    pos = positional[0, :seq, :]
    pos2 = jnp.concatenate([pos, pos], axis=0)
    info = plsc.get_sparse_core_info()
    out_flat = _embed_kernel(idx_flat.shape[0], info.num_cores, info.num_subcores)(
        idx_flat, pos2, embedding)
    return out_flat.reshape(batch, seq, D)


# speedup vs baseline: 4.0124x; 2.2886x over previous
import functools

import jax
import jax.numpy as jnp
from jax import lax
from jax.experimental import pallas as pl
from jax.experimental.pallas import tpu as pltpu
from jax.experimental.pallas import tpu_sc as plsc

D = 128
SEQ = 200
CH = 80
NBUF = 5
MAXP0 = max((CH * b) % SEQ for b in range(NBUF))
POS_ROWS = MAXP0 + CH
LANES = 16


def _embed_kernel(T, NC, NS):
    NW = NC * NS
    tok_w = T // NW
    nchunk = tok_w // CH
    ngroup = nchunk // NBUF
    mesh = plsc.VectorSubcoreMesh(core_axis_name="c", subcore_axis_name="s")

    @functools.partial(
        pl.kernel,
        mesh=mesh,
        out_type=jax.ShapeDtypeStruct((T, D), jnp.float32),
        scratch_types=[
            pltpu.VMEM((nchunk, CH), jnp.int32),
            pltpu.VMEM((POS_ROWS, D), jnp.float32),
            pltpu.VMEM((NBUF, CH, D), jnp.float32),
        ] + [pltpu.SemaphoreType.DMA] * (2 * NBUF),
    )
    def k(idx_hbm, pos_hbm, table_hbm, out_hbm, idx_v, pos_v, rows_v, *sems):
        gsem = sems[:NBUF]
        osem = sems[NBUF:]
        c = lax.axis_index("c")
        s = lax.axis_index("s")
        wid = s * NC + c
        base = wid * tok_w

        pltpu.sync_copy(idx_hbm.at[pl.ds(wid * nchunk, nchunk)], idx_v)
        pltpu.sync_copy(pos_hbm, pos_v)
        pltpu.async_copy(table_hbm.at[idx_v.at[0]], rows_v.at[0], gsem[0])

        def group_body(g, carry):
            for b in range(NBUF):
                ci = g * NBUF + b
                nslot = (b + 1) % NBUF

                def wait_out():
                    pltpu.make_async_copy(
                        rows_v.at[nslot], out_hbm.at[pl.ds(0, CH)],
                        osem[nslot]).wait()
                if b == NBUF - 1:
                    wait_out()
                else:
                    pl.when(g >= 1)(wait_out)

                def issue_gather():
                    pltpu.async_copy(
                        table_hbm.at[idx_v.at[ci + 1]], rows_v.at[nslot],
                        gsem[nslot])
                if b == NBUF - 1:
                    pl.when(g < ngroup - 1)(issue_gather)
                else:
                    issue_gather()

                pltpu.make_async_copy(
                    table_hbm.at[idx_v.at[0]], rows_v.at[b], gsem[b]).wait()

                p0 = (CH * b) % SEQ

                def row_body(r, rc):
                    for u in range(2):
                        rr = r * 2 + u
                        for j in range(D // LANES):
                            sl = pl.ds(j * LANES, LANES)
                            plsc.addupdate(rows_v.at[b, rr, sl],
                                           pos_v[p0 + rr, sl])
                    return rc

                lax.fori_loop(0, CH // 2, row_body, 0)

                pltpu.async_copy(
                    rows_v.at[b], out_hbm.at[pl.ds(base + ci * CH, CH)],
                    osem[b])
            return carry

        lax.fori_loop(0, ngroup, group_body, 0)

        for b in range(1, NBUF):
            pltpu.make_async_copy(
                rows_v.at[b], out_hbm.at[pl.ds(0, CH)], osem[b]).wait()

    return k


def kernel(code_bytes, embedding, positional):
    batch, seq = code_bytes.shape
    T = batch * seq
    idx2d = code_bytes.reshape(T // CH, CH).astype(jnp.int32)
    pos = positional[0, :seq, :]
    pos2 = jnp.concatenate([pos, pos[:POS_ROWS - seq]], axis=0)
    info = plsc.get_sparse_core_info()
    out_flat = _embed_kernel(T, info.num_cores, info.num_subcores)(
        idx2d, pos2, embedding)
    return out_flat.reshape(batch, seq, D)


# speedup vs baseline: 4.0998x; 1.0218x over previous
import functools

import jax
import jax.numpy as jnp
from jax import lax
from jax.experimental import pallas as pl
from jax.experimental.pallas import tpu as pltpu
from jax.experimental.pallas import tpu_sc as plsc

D = 128
SEQ = 200
CH = 80
NBUF = 5
MAXP0 = max((CH * b) % SEQ for b in range(NBUF))
POS_ROWS = MAXP0 + CH
LANES = 16


def _embed_kernel(T, NC, NS):
    NW = NC * NS
    tok_w = T // NW
    nchunk = tok_w // CH
    ngroup = nchunk // NBUF
    mesh = plsc.VectorSubcoreMesh(core_axis_name="c", subcore_axis_name="s")

    @functools.partial(
        pl.kernel,
        mesh=mesh,
        out_type=jax.ShapeDtypeStruct((T, D), jnp.float32),
        scratch_types=[
            pltpu.VMEM((nchunk, CH), jnp.int32),
            pltpu.VMEM((POS_ROWS, D), jnp.float32),
            pltpu.VMEM((NBUF, CH, D), jnp.float32),
        ] + [pltpu.SemaphoreType.DMA] * (2 * NBUF),
    )
    def k(idx_hbm, pos_hbm, table_hbm, out_hbm, idx_v, pos_v, rows_v, *sems):
        gsem = sems[:NBUF]
        osem = sems[NBUF:]
        c = lax.axis_index("c")
        s = lax.axis_index("s")
        wid = s * NC + c
        base = wid * tok_w

        PF = 3
        pltpu.sync_copy(idx_hbm.at[pl.ds(wid * nchunk, nchunk)], idx_v)
        pltpu.sync_copy(pos_hbm, pos_v)
        for j in range(PF):
            pltpu.async_copy(table_hbm.at[idx_v.at[j]], rows_v.at[j], gsem[j])

        def group_body(g, carry):
            for b in range(NBUF):
                ci = g * NBUF + b
                pslot = (b + PF) % NBUF

                def wait_out():
                    pltpu.make_async_copy(
                        rows_v.at[pslot], out_hbm.at[pl.ds(0, CH)],
                        osem[pslot]).wait()
                if b >= NBUF - PF:
                    wait_out()
                else:
                    pl.when(g >= 1)(wait_out)

                def issue_gather():
                    pltpu.async_copy(
                        table_hbm.at[idx_v.at[ci + PF]], rows_v.at[pslot],
                        gsem[pslot])
                pl.when(ci + PF < nchunk)(issue_gather)

                pltpu.make_async_copy(
                    table_hbm.at[idx_v.at[0]], rows_v.at[b], gsem[b]).wait()

                p0 = (CH * b) % SEQ

                def row_body(r, rc):
                    for u in range(2):
                        rr = r * 2 + u
                        for j in range(D // LANES):
                            sl = pl.ds(j * LANES, LANES)
                            plsc.addupdate(rows_v.at[b, rr, sl],
                                           pos_v[p0 + rr, sl])
                    return rc

                lax.fori_loop(0, CH // 2, row_body, 0)

                pltpu.async_copy(
                    rows_v.at[b], out_hbm.at[pl.ds(base + ci * CH, CH)],
                    osem[b])
            return carry

        lax.fori_loop(0, ngroup, group_body, 0)

        for ci in range(nchunk - (NBUF - PF), nchunk):
            pltpu.make_async_copy(
                rows_v.at[ci % NBUF], out_hbm.at[pl.ds(0, CH)],
                osem[ci % NBUF]).wait()

    return k


def kernel(code_bytes, embedding, positional):
    batch, seq = code_bytes.shape
    T = batch * seq
    idx2d = code_bytes.reshape(T // CH, CH).astype(jnp.int32)
    pos = positional[0, :seq, :]
    pos2 = jnp.concatenate([pos, pos[:POS_ROWS - seq]], axis=0)
    info = plsc.get_sparse_core_info()
    out_flat = _embed_kernel(T, info.num_cores, info.num_subcores)(
        idx2d, pos2, embedding)
    return out_flat.reshape(batch, seq, D)
